# Initial kernel scaffold; baseline (speedup 1.0000x reference)
import jax
import jax.numpy as jnp
from jax.experimental import pallas as pl


def kernel(hidden_states, gate_w, w_gate, w_up, w_down):
    raise NotImplementedError("write your pallas kernel here")


# speedup vs baseline: 1.8877x; 1.8877x over previous
import functools

import jax
import jax.numpy as jnp
from jax import lax
from jax.experimental import pallas as pl
from jax.experimental.pallas import tpu as pltpu

_E = 8
_H = 1024
_I = 512
_ZC = 0.01
_TT = 1024


def _router_body(x_ref, gw_ref, logits_ref, scores_ref):
    x = x_ref[...]
    gw = gw_ref[...]
    logits = lax.dot_general(
        x, gw, (((1,), (1,)), ((), ())), preferred_element_type=jnp.float32
    )
    logits_ref[...] = logits
    m = jnp.mean(logits, axis=-1, keepdims=True)
    c = logits - m
    z_loss = _ZC * jnp.sum(c * c) / (logits.shape[0] * logits.shape[1])
    l2 = logits - z_loss * logits
    rowmax = jnp.max(l2, axis=-1, keepdims=True)
    p = jnp.exp(l2 - rowmax)
    p = p / jnp.sum(p, axis=-1, keepdims=True)
    scores_ref[...] = p / jnp.clip(jnp.sum(p, axis=-1, keepdims=True), 1e-8, None)


def _expert_body(x_ref, s_ref, wg_ref, wu_ref, wd_ref, out_ref):
    e = pl.program_id(1)
    x = x_ref[...]
    g = lax.dot_general(
        x, wg_ref[0], (((1,), (1,)), ((), ())), preferred_element_type=jnp.float32
    )
    u = lax.dot_general(
        x, wu_ref[0], (((1,), (1,)), ((), ())), preferred_element_type=jnp.float32
    )
    h = g * jax.nn.sigmoid(g) * u
    lane = lax.broadcasted_iota(jnp.int32, (1, _E), 1)
    s_col = jnp.sum(
        jnp.where(lane == e, s_ref[...], 0.0), axis=-1, keepdims=True
    )
    h = h * s_col
    o = lax.dot_general(
        h, wd_ref[0], (((1,), (1,)), ((), ())), preferred_element_type=jnp.float32
    )

    @pl.when(e == 0)
    def _():
        out_ref[...] = o

    @pl.when(e != 0)
    def _():
        out_ref[...] += o


@jax.jit
def kernel(hidden_states, gate_w, w_gate, w_up, w_down):
    B, S, H = hidden_states.shape
    T = B * S
    x = hidden_states.reshape(T, H)

    router_logits, scores = pl.pallas_call(
        _router_body,
        out_shape=(
            jax.ShapeDtypeStruct((T, _E), jnp.float32),
            jax.ShapeDtypeStruct((T, _E), jnp.float32),
        ),
    )(x, gate_w)

    n_t = T // _TT
    final = pl.pallas_call(
        _expert_body,
        grid=(n_t, _E),
        in_specs=[
            pl.BlockSpec((_TT, _H), lambda t, e: (t, 0)),
            pl.BlockSpec((_TT, _E), lambda t, e: (t, 0)),
            pl.BlockSpec((1, _I, _H), lambda t, e: (e, 0, 0)),
            pl.BlockSpec((1, _I, _H), lambda t, e: (e, 0, 0)),
            pl.BlockSpec((1, _H, _I), lambda t, e: (e, 0, 0)),
        ],
        out_specs=pl.BlockSpec((_TT, _H), lambda t, e: (t, 0)),
        out_shape=jax.ShapeDtypeStruct((T, _H), jnp.float32),
        compiler_params=pltpu.CompilerParams(
            dimension_semantics=("parallel", "arbitrary"),
        ),
    )(x, scores, w_gate, w_up, w_down)

    return final.reshape(B, S, H), router_logits
